# baseline (speedup 1.0000x reference)
import functools

import jax
import jax.numpy as jnp
from jax import lax
from jax.experimental import pallas as pl
from jax.experimental.pallas import tpu as pltpu
from jax.experimental.pallas import tpu_sc as plsc

N = 10000
E = 160000
F = 256
G = 16
FG = 16
NC = 2
NS = 16
HP = E // 2
CH = 128
NCH = 5
BK = CH * NCH
NBLK = HP // BK
ROWS_PER_SC = (G // NC) * N
FIX_ROWS = ROWS_PER_SC // NS
FIX_BLK = 200
RS_PAD = 208
ALPHA = 0.2
INV_SQRT_F = 1.0 / 16.0


def _matmul_tc(h, W, b, a1, a2):
    RB = 2000
    grid = (N // RB,)

    def body(h_ref, w_ref, b_ref, a1_ref, a2_ref, data_ref, s1_ref, s2_ref):
        dat = lax.dot_general(h_ref[...], w_ref[...],
                              (((1,), (1,)), ((), ())),
                              preferred_element_type=jnp.float32)
        dat = dat + b_ref[...]
        data_ref[...] = dat
        s1_ref[...] = lax.dot_general(dat, a1_ref[...],
                                      (((1,), (0,)), ((), ())),
                                      preferred_element_type=jnp.float32)
        s2_ref[...] = lax.dot_general(dat, a2_ref[...],
                                      (((1,), (0,)), ((), ())),
                                      preferred_element_type=jnp.float32)

    return pl.pallas_call(
        body,
        grid=grid,
        in_specs=[
            pl.BlockSpec((RB, F), lambda i: (i, 0)),
            pl.BlockSpec((F, F), lambda i: (0, 0)),
            pl.BlockSpec((1, F), lambda i: (0, 0)),
            pl.BlockSpec((F, 1), lambda i: (0, 0)),
            pl.BlockSpec((F, 1), lambda i: (0, 0)),
        ],
        out_specs=[
            pl.BlockSpec((RB, F), lambda i: (i, 0)),
            pl.BlockSpec((RB, 1), lambda i: (i, 0)),
            pl.BlockSpec((RB, 1), lambda i: (i, 0)),
        ],
        out_shape=[
            jax.ShapeDtypeStruct((N, F), jnp.float32),
            jax.ShapeDtypeStruct((N, 1), jnp.float32),
            jax.ShapeDtypeStruct((N, 1), jnp.float32),
        ],
    )(h, W, b.reshape(1, F), a1, a2)


def _sc_spmm(data_flat, src, dst, s1, s2):
    mesh = plsc.VectorSubcoreMesh(core_axis_name="c", subcore_axis_name="s",
                                  num_cores=NC, num_subcores=NS)

    @functools.partial(
        pl.kernel,
        out_type=jax.ShapeDtypeStruct((N, F), jnp.float32),
        mesh=mesh,
        compiler_params=pltpu.CompilerParams(needs_layout_passes=False,
                                             use_tc_tiling_on_sc=False),
        scratch_types=[
            pltpu.VMEM((N,), jnp.float32),
            pltpu.VMEM((N,), jnp.float32),
            pltpu.VMEM((BK,), jnp.int32),
            pltpu.VMEM((BK,), jnp.int32),
            pltpu.VMEM((2, NCH, CH), jnp.int32),
            pltpu.VMEM((2, NCH, CH), jnp.int32),
            pltpu.VMEM((2, NCH, CH), jnp.int32),
            pltpu.VMEM((2, BK), jnp.float32),
            pltpu.VMEM((2, BK, FG), jnp.float32),
            pltpu.SemaphoreType.DMA,
            pltpu.SemaphoreType.DMA,
            pltpu.SemaphoreType.DMA,
            [pltpu.SemaphoreType.DMA] * NCH,
            pltpu.VMEM_SHARED((ROWS_PER_SC, FG), jnp.float32),
            pltpu.VMEM_SHARED((2 * N,), jnp.float32),
        ],
    )
    def k(data_hbm, src_hbm, dst_hbm, s1_hbm, s2_hbm, out_hbm,
          s1_v, s2_v, src_v, dst_v, gidx_v, sidx_v, ridx_v, e_v, rows_v,
          sem_l, sem_s, sem_r, sem_g, acc_sh, rs_sh):
        c = lax.axis_index("c")
        s = lax.axis_index("s")
        gl = s // 2
        g = c * (G // NC) + gl
        half = s % 2
        e0 = half * HP

        zero16 = jnp.zeros((FG,), jnp.float32)
        _LANE = [jnp.full((16,), j, jnp.int32) for j in range(16)]

        pltpu.sync_copy(s1_hbm, s1_v)
        pltpu.sync_copy(s2_hbm, s2_v)

        def z_ab(j, _):
            rows_v[0, j, :] = zero16
            return 0
        lax.fori_loop(0, BK, z_ab, 0)
        zbase = gl * N + half * (N // 2)

        def z_acc(kk, _):
            pltpu.sync_copy(rows_v.at[0],
                            acc_sh.at[pl.ds(zbase + kk * BK, BK), :])
            return 0
        lax.fori_loop(0, (N // 2) // BK, z_acc, 0)
        pltpu.sync_copy(rows_v.at[0, pl.ds(0, (N // 2) % BK), :],
                        acc_sh.at[pl.ds(zbase + ((N // 2) // BK) * BK,
                                        (N // 2) % BK), :])

        @pl.when(gl == 0)
        def _z_rs():
            def z_e(j, _):
                e_v[0, pl.ds(j * 16, 16)] = zero16
                return 0
            lax.fori_loop(0, BK // 16, z_e, 0)

            def z_rsh(kk, _):
                pltpu.sync_copy(e_v.at[0],
                                rs_sh.at[pl.ds(half * N + kk * BK, BK)])
                return 0
            lax.fori_loop(0, N // BK, z_rsh, 0)
            pltpu.sync_copy(e_v.at[0, pl.ds(0, N % BK)],
                            rs_sh.at[pl.ds(half * N + (N // BK) * BK, N % BK)])

        plsc.subcore_barrier()

        def fire_loads(j):
            jc = jnp.minimum(j, NBLK - 1)
            base = e0 + jc * BK
            pltpu.async_copy(src_hbm.at[pl.ds(base, BK)], src_v, sem_l)
            pltpu.async_copy(dst_hbm.at[pl.ds(base, BK)], dst_v, sem_l)

        def drain_loads():
            pltpu.make_async_copy(src_hbm.at[pl.ds(0, BK)], src_v, sem_l).wait()
            pltpu.make_async_copy(dst_hbm.at[pl.ds(0, BK)], dst_v, sem_l).wait()

        def grp_compute(q):
            @plsc.parallel_loop(0, BK // 16, unroll=2)
            def _grp(i):
                sl = pl.ds(i * 16, 16)
                s16 = src_v[sl]
                d16 = dst_v[sl]
                sc = plsc.load_gather(s1_v, [s16]) + plsc.load_gather(s2_v, [d16])
                m = jnp.maximum(sc, sc * ALPHA)
                e16 = jnp.exp(m * INV_SQRT_F)
                e_v[q, sl] = e16
                kkq = i // 8
                lsl = pl.ds((i % 8) * 16, 16)
                gidx_v[q, kkq, lsl] = d16 + g * N
                sidx_v[q, kkq, lsl] = s16 + gl * N
                ridx_v[q, kkq, lsl] = s16 + half * N

        def fire_gathers(q):
            for kk in range(NCH):
                pltpu.async_copy(data_hbm.at[gidx_v.at[q, kk]],
                                 rows_v.at[q, pl.ds(kk * CH, CH), :],
                                 sem_g[kk])

        fire_loads(0)
        drain_loads()
        grp_compute(0)
        fire_loads(1)
        fire_gathers(0)

        def block(bi, _):
            p = bi % 2
            q = 1 - p
            drain_loads()
            grp_compute(q)
            fire_loads(bi + 2)

            for kk in range(NCH):
                pltpu.make_async_copy(
                    data_hbm.at[gidx_v.at[p, kk]],
                    rows_v.at[p, pl.ds(kk * CH, CH), :],
                    sem_g[kk]).wait()

                @plsc.parallel_loop(0, CH // 16, unroll=4)
                def _scale(i):
                    e16 = e_v[p, pl.ds(kk * CH + i * 16, 16)]
                    r0 = kk * CH + i * 16
                    for j2 in range(16):
                        mult = e16.at[_LANE[j2]].get(mode="promise_in_bounds")
                        rows_v[p, r0 + j2, :] = rows_v[p, r0 + j2, :] * mult

                pltpu.async_copy(rows_v.at[p, pl.ds(kk * CH, CH), :],
                                 acc_sh.at[sidx_v.at[p, kk]],
                                 sem_s, add=True)

                @pl.when(gl == 0)
                def _rs_add():
                    pltpu.async_copy(e_v.at[p, pl.ds(kk * CH, CH)],
                                     rs_sh.at[ridx_v.at[p, kk]],
                                     sem_r, add=True)

            @pl.when(bi < NBLK - 1)
            def _next_gathers():
                fire_gathers(q)

            for kk in range(NCH):
                pltpu.make_async_copy(rows_v.at[p, pl.ds(kk * CH, CH), :],
                                      acc_sh.at[sidx_v.at[p, kk]],
                                      sem_s).wait()

            @pl.when(gl == 0)
            def _rs_drain():
                for kk in range(NCH):
                    pltpu.make_async_copy(e_v.at[p, pl.ds(kk * CH, CH)],
                                          rs_sh.at[ridx_v.at[p, kk]],
                                          sem_r).wait()
            return 0
        lax.fori_loop(0, NBLK, block, 0)

        drain_loads()

        plsc.subcore_barrier()

        rbase0 = s * FIX_ROWS

        def fixblk(blk, _):
            rb = rbase0 + blk * FIX_BLK
            pltpu.sync_copy(acc_sh.at[pl.ds(rb, FIX_BLK), :],
                            rows_v.at[0, pl.ds(0, FIX_BLK), :])
            pltpu.sync_copy(data_hbm.at[pl.ds(c * ROWS_PER_SC + rb, FIX_BLK), :],
                            rows_v.at[1, pl.ds(0, FIX_BLK), :])
            nb = (rbase0 + blk * FIX_BLK) % N
            pltpu.sync_copy(rs_sh.at[pl.ds(nb, FIX_BLK)],
                            e_v.at[0, pl.ds(0, FIX_BLK)])
            pltpu.sync_copy(rs_sh.at[pl.ds(N + nb, FIX_BLK)],
                            e_v.at[1, pl.ds(0, FIX_BLK)])

            def rspass(i, _):
                sl = pl.ds(i * 16, 16)
                t = e_v[0, sl] + e_v[1, sl]
                iszero = t == 0.0
                den = jnp.where(iszero, 1.0, t)
                e_v[0, sl] = 1.0 / den
                e_v[1, sl] = jnp.where(iszero, 1.0, 0.0)
                return 0
            lax.fori_loop(0, RS_PAD // 16, rspass, 0)

            def rowfix(i, _):
                rcp16 = e_v[0, pl.ds(i * 16, 16)]
                m16 = e_v[1, pl.ds(i * 16, 16)]
                r0 = i * 16
                for j2 in range(16):
                    rcp = rcp16.at[_LANE[j2]].get(mode="promise_in_bounds")
                    m = m16.at[_LANE[j2]].get(mode="promise_in_bounds")
                    rows_v[0, r0 + j2, :] = (rows_v[0, r0 + j2, :]
                                             + m * rows_v[1, r0 + j2, :]) * rcp
                return 0
            lax.fori_loop(0, FIX_BLK // 16, rowfix, 0)

            t0 = (FIX_BLK // 16) * 16
            rcp16 = e_v[0, pl.ds(t0, 16)]
            m16 = e_v[1, pl.ds(t0, 16)]
            for j2 in range(FIX_BLK - t0):
                rcp = rcp16.at[_LANE[j2]].get(mode="promise_in_bounds")
                m = m16.at[_LANE[j2]].get(mode="promise_in_bounds")
                rows_v[0, t0 + j2, :] = (rows_v[0, t0 + j2, :]
                                         + m * rows_v[1, t0 + j2, :]) * rcp
            pltpu.sync_copy(rows_v.at[0, pl.ds(0, FIX_BLK), :],
                            out_hbm.at[pl.ds(nb, FIX_BLK),
                                       pl.ds(g * FG, FG)])
            return 0
        lax.fori_loop(0, FIX_ROWS // FIX_BLK, fixblk, 0)

    return k(data_flat, src, dst, s1, s2)


def kernel(h, adj, W, b, a):
    src = adj[0].astype(jnp.int32)
    dst = adj[1].astype(jnp.int32)
    a1 = a[0, :F].reshape(F, 1)
    a2 = a[0, F:].reshape(F, 1)

    data, s1, s2 = _matmul_tc(h, W, b, a1, a2)

    data_flat = data.reshape(N, G, FG).transpose(1, 0, 2).reshape(G * N, FG)

    return _sc_spmm(data_flat, src, dst, s1.reshape(N), s2.reshape(N))


# speedup vs baseline: 1.0882x; 1.0882x over previous
import functools

import jax
import jax.numpy as jnp
from jax import lax
from jax.experimental import pallas as pl
from jax.experimental.pallas import tpu as pltpu
from jax.experimental.pallas import tpu_sc as plsc

N = 10000
E = 160000
F = 256
G = 16
FG = 16
NC = 2
NS = 16
HP = E // 2
CH = 128
NCH = 5
BK = CH * NCH
NBLK = HP // BK
ROWS_PER_SC = (G // NC) * N
FIX_ROWS = ROWS_PER_SC // NS
FIX_BLK = 200
RS_PAD = 208
ALPHA = 0.2
INV_SQRT_F = 1.0 / 16.0


def _matmul_tc(h, W, b, a1, a2):
    RB = 2000
    grid = (N // RB,)

    def body(h_ref, w_ref, b_ref, a1_ref, a2_ref, data_ref, s1_ref, s2_ref):
        dat = lax.dot_general(h_ref[...], w_ref[...],
                              (((1,), (1,)), ((), ())),
                              preferred_element_type=jnp.float32)
        dat = dat + b_ref[...]
        data_ref[...] = dat
        s1_ref[...] = lax.dot_general(dat, a1_ref[...],
                                      (((1,), (0,)), ((), ())),
                                      preferred_element_type=jnp.float32)
        s2_ref[...] = lax.dot_general(dat, a2_ref[...],
                                      (((1,), (0,)), ((), ())),
                                      preferred_element_type=jnp.float32)

    return pl.pallas_call(
        body,
        grid=grid,
        in_specs=[
            pl.BlockSpec((RB, F), lambda i: (i, 0)),
            pl.BlockSpec((F, F), lambda i: (0, 0)),
            pl.BlockSpec((1, F), lambda i: (0, 0)),
            pl.BlockSpec((F, 1), lambda i: (0, 0)),
            pl.BlockSpec((F, 1), lambda i: (0, 0)),
        ],
        out_specs=[
            pl.BlockSpec((RB, F), lambda i: (i, 0)),
            pl.BlockSpec((RB, 1), lambda i: (i, 0)),
            pl.BlockSpec((RB, 1), lambda i: (i, 0)),
        ],
        out_shape=[
            jax.ShapeDtypeStruct((N, F), jnp.float32),
            jax.ShapeDtypeStruct((N, 1), jnp.float32),
            jax.ShapeDtypeStruct((N, 1), jnp.float32),
        ],
    )(h, W, b.reshape(1, F), a1, a2)


def _sc_spmm(data_flat, src, dst, s1, s2):
    mesh = plsc.VectorSubcoreMesh(core_axis_name="c", subcore_axis_name="s",
                                  num_cores=NC, num_subcores=NS)

    @functools.partial(
        pl.kernel,
        out_type=jax.ShapeDtypeStruct((N, F), jnp.float32),
        mesh=mesh,
        compiler_params=pltpu.CompilerParams(needs_layout_passes=False,
                                             use_tc_tiling_on_sc=False),
        scratch_types=[
            pltpu.VMEM((N,), jnp.float32),
            pltpu.VMEM((N,), jnp.float32),
            pltpu.VMEM((BK,), jnp.int32),
            pltpu.VMEM((BK,), jnp.int32),
            pltpu.VMEM((2, NCH, CH), jnp.int32),
            pltpu.VMEM((2, NCH, CH), jnp.int32),
            pltpu.VMEM((2, NCH, CH), jnp.int32),
            pltpu.VMEM((2, BK), jnp.float32),
            pltpu.VMEM((2, BK, FG), jnp.float32),
            pltpu.SemaphoreType.DMA,
            pltpu.SemaphoreType.DMA,
            pltpu.SemaphoreType.DMA,
            [pltpu.SemaphoreType.DMA] * NCH,
            pltpu.VMEM_SHARED((ROWS_PER_SC, FG), jnp.float32),
            pltpu.VMEM_SHARED((2 * N,), jnp.float32),
        ],
    )
    def k(data_hbm, src_hbm, dst_hbm, s1_hbm, s2_hbm, out_hbm,
          s1_v, s2_v, src_v, dst_v, gidx_v, sidx_v, ridx_v, e_v, rows_v,
          sem_l, sem_s, sem_r, sem_g, acc_sh, rs_sh):
        c = lax.axis_index("c")
        s = lax.axis_index("s")
        gl = s // 2
        g = c * (G // NC) + gl
        half = s % 2
        e0 = half * HP

        zero16 = jnp.zeros((FG,), jnp.float32)
        _LANE = [jnp.full((16,), j, jnp.int32) for j in range(16)]

        pltpu.sync_copy(s1_hbm, s1_v)
        pltpu.sync_copy(s2_hbm, s2_v)

        def z_ab(j, _):
            rows_v[0, j, :] = zero16
            return 0
        lax.fori_loop(0, BK, z_ab, 0)
        zbase = gl * N + half * (N // 2)

        def z_acc(kk, _):
            pltpu.sync_copy(rows_v.at[0],
                            acc_sh.at[pl.ds(zbase + kk * BK, BK), :])
            return 0
        lax.fori_loop(0, (N // 2) // BK, z_acc, 0)
        pltpu.sync_copy(rows_v.at[0, pl.ds(0, (N // 2) % BK), :],
                        acc_sh.at[pl.ds(zbase + ((N // 2) // BK) * BK,
                                        (N // 2) % BK), :])

        @pl.when(gl == 0)
        def _z_rs():
            def z_e(j, _):
                e_v[0, pl.ds(j * 16, 16)] = zero16
                return 0
            lax.fori_loop(0, BK // 16, z_e, 0)

            def z_rsh(kk, _):
                pltpu.sync_copy(e_v.at[0],
                                rs_sh.at[pl.ds(half * N + kk * BK, BK)])
                return 0
            lax.fori_loop(0, N // BK, z_rsh, 0)
            pltpu.sync_copy(e_v.at[0, pl.ds(0, N % BK)],
                            rs_sh.at[pl.ds(half * N + (N // BK) * BK, N % BK)])

        plsc.subcore_barrier()

        def fire_loads(j):
            jc = jnp.minimum(j, NBLK - 1)
            base = e0 + jc * BK
            pltpu.async_copy(src_hbm.at[pl.ds(base, BK)], src_v, sem_l)
            pltpu.async_copy(dst_hbm.at[pl.ds(base, BK)], dst_v, sem_l)

        def drain_loads():
            pltpu.make_async_copy(src_hbm.at[pl.ds(0, BK)], src_v, sem_l).wait()
            pltpu.make_async_copy(dst_hbm.at[pl.ds(0, BK)], dst_v, sem_l).wait()

        def grp_compute(q):
            @plsc.parallel_loop(0, BK // 16, unroll=2)
            def _grp(i):
                sl = pl.ds(i * 16, 16)
                s16 = src_v[sl]
                d16 = dst_v[sl]
                sc = plsc.load_gather(s1_v, [s16]) + plsc.load_gather(s2_v, [d16])
                m = jnp.maximum(sc, sc * ALPHA)
                e16 = jnp.exp(m * INV_SQRT_F)
                e_v[q, sl] = e16
                kkq = i // 8
                lsl = pl.ds((i % 8) * 16, 16)
                gidx_v[q, kkq, lsl] = d16 + g * N
                sidx_v[q, kkq, lsl] = s16 + gl * N
                ridx_v[q, kkq, lsl] = s16 + half * N

        def fire_gathers(q):
            for kk in range(NCH):
                pltpu.async_copy(data_hbm.at[gidx_v.at[q, kk]],
                                 rows_v.at[q, pl.ds(kk * CH, CH), :],
                                 sem_g[kk])

        fire_loads(0)
        drain_loads()
        grp_compute(0)
        fire_loads(1)
        fire_gathers(0)

        def block(bi, _):
            p = bi % 2
            q = 1 - p
            drain_loads()
            grp_compute(q)
            fire_loads(bi + 2)

            for kk in range(NCH):
                pltpu.make_async_copy(
                    data_hbm.at[gidx_v.at[p, kk]],
                    rows_v.at[p, pl.ds(kk * CH, CH), :],
                    sem_g[kk]).wait()

                @plsc.parallel_loop(0, CH // 16, unroll=2)
                def _scale(i):
                    e16 = e_v[p, pl.ds(kk * CH + i * 16, 16)]
                    r0 = kk * CH + i * 16
                    for j2 in range(16):
                        mult = e16.at[_LANE[j2]].get(mode="promise_in_bounds")
                        rows_v[p, r0 + j2, :] = rows_v[p, r0 + j2, :] * mult

                pltpu.async_copy(rows_v.at[p, pl.ds(kk * CH, CH), :],
                                 acc_sh.at[sidx_v.at[p, kk]],
                                 sem_s, add=True)

                @pl.when(gl == bi % 8)
                def _rs_add():
                    pltpu.async_copy(e_v.at[p, pl.ds(kk * CH, CH)],
                                     rs_sh.at[ridx_v.at[p, kk]],
                                     sem_r, add=True)

            @pl.when(bi < NBLK - 1)
            def _next_gathers():
                fire_gathers(q)

            for kk in range(NCH):
                pltpu.make_async_copy(rows_v.at[p, pl.ds(kk * CH, CH), :],
                                      acc_sh.at[sidx_v.at[p, kk]],
                                      sem_s).wait()

            @pl.when(gl == bi % 8)
            def _rs_drain():
                for kk in range(NCH):
                    pltpu.make_async_copy(e_v.at[p, pl.ds(kk * CH, CH)],
                                          rs_sh.at[ridx_v.at[p, kk]],
                                          sem_r).wait()
            return 0
        lax.fori_loop(0, NBLK, block, 0)

        drain_loads()

        plsc.subcore_barrier()

        rbase0 = s * FIX_ROWS

        def fixblk(blk, _):
            rb = rbase0 + blk * FIX_BLK
            pltpu.sync_copy(acc_sh.at[pl.ds(rb, FIX_BLK), :],
                            rows_v.at[0, pl.ds(0, FIX_BLK), :])
            pltpu.sync_copy(data_hbm.at[pl.ds(c * ROWS_PER_SC + rb, FIX_BLK), :],
                            rows_v.at[1, pl.ds(0, FIX_BLK), :])
            nb = (rbase0 + blk * FIX_BLK) % N
            pltpu.sync_copy(rs_sh.at[pl.ds(nb, FIX_BLK)],
                            e_v.at[0, pl.ds(0, FIX_BLK)])
            pltpu.sync_copy(rs_sh.at[pl.ds(N + nb, FIX_BLK)],
                            e_v.at[1, pl.ds(0, FIX_BLK)])

            def rspass(i, _):
                sl = pl.ds(i * 16, 16)
                t = e_v[0, sl] + e_v[1, sl]
                iszero = t == 0.0
                den = jnp.where(iszero, 1.0, t)
                e_v[0, sl] = 1.0 / den
                e_v[1, sl] = jnp.where(iszero, 1.0, 0.0)
                return 0
            lax.fori_loop(0, RS_PAD // 16, rspass, 0)

            @plsc.parallel_loop(0, FIX_BLK // 16, unroll=2)
            def _rowfix(i):
                rcp16 = e_v[0, pl.ds(i * 16, 16)]
                m16 = e_v[1, pl.ds(i * 16, 16)]
                r0 = i * 16
                for j2 in range(16):
                    rcp = rcp16.at[_LANE[j2]].get(mode="promise_in_bounds")
                    m = m16.at[_LANE[j2]].get(mode="promise_in_bounds")
                    rows_v[0, r0 + j2, :] = (rows_v[0, r0 + j2, :]
                                             + m * rows_v[1, r0 + j2, :]) * rcp

            t0 = (FIX_BLK // 16) * 16
            rcp16 = e_v[0, pl.ds(t0, 16)]
            m16 = e_v[1, pl.ds(t0, 16)]
            for j2 in range(FIX_BLK - t0):
                rcp = rcp16.at[_LANE[j2]].get(mode="promise_in_bounds")
                m = m16.at[_LANE[j2]].get(mode="promise_in_bounds")
                rows_v[0, t0 + j2, :] = (rows_v[0, t0 + j2, :]
                                         + m * rows_v[1, t0 + j2, :]) * rcp
            pltpu.sync_copy(rows_v.at[0, pl.ds(0, FIX_BLK), :],
                            out_hbm.at[pl.ds(nb, FIX_BLK),
                                       pl.ds(g * FG, FG)])
            return 0
        lax.fori_loop(0, FIX_ROWS // FIX_BLK, fixblk, 0)

    return k(data_flat, src, dst, s1, s2)


def kernel(h, adj, W, b, a):
    src = adj[0].astype(jnp.int32)
    dst = adj[1].astype(jnp.int32)
    a1 = a[0, :F].reshape(F, 1)
    a2 = a[0, F:].reshape(F, 1)

    data, s1, s2 = _matmul_tc(h, W, b, a1, a2)

    data_flat = data.reshape(N, G, FG).transpose(1, 0, 2).reshape(G * N, FG)

    return _sc_spmm(data_flat, src, dst, s1.reshape(N), s2.reshape(N))


# speedup vs baseline: 1.0894x; 1.0011x over previous
import functools

import jax
import jax.numpy as jnp
from jax import lax
from jax.experimental import pallas as pl
from jax.experimental.pallas import tpu as pltpu
from jax.experimental.pallas import tpu_sc as plsc

N = 10000
E = 160000
F = 256
G = 16
FG = 16
NC = 2
NS = 16
HP = E // 2
CH = 128
NCH = 5
BK = CH * NCH
NBLK = HP // BK
ROWS_PER_SC = (G // NC) * N
FIX_ROWS = ROWS_PER_SC // NS
FIX_BLK = 200
RS_PAD = 208
ALPHA = 0.2
INV_SQRT_F = 1.0 / 16.0


def _matmul_tc(h, W, b, a1, a2):
    RB = 2000
    grid = (N // RB,)

    def body(h_ref, w_ref, b_ref, a1_ref, a2_ref, data_ref, s1_ref, s2_ref):
        dat = lax.dot_general(h_ref[...], w_ref[...],
                              (((1,), (1,)), ((), ())),
                              preferred_element_type=jnp.float32)
        dat = dat + b_ref[...]
        data_ref[...] = dat
        s1_ref[...] = lax.dot_general(dat, a1_ref[...],
                                      (((1,), (0,)), ((), ())),
                                      preferred_element_type=jnp.float32)
        s2_ref[...] = lax.dot_general(dat, a2_ref[...],
                                      (((1,), (0,)), ((), ())),
                                      preferred_element_type=jnp.float32)

    return pl.pallas_call(
        body,
        grid=grid,
        in_specs=[
            pl.BlockSpec((RB, F), lambda i: (i, 0)),
            pl.BlockSpec((F, F), lambda i: (0, 0)),
            pl.BlockSpec((1, F), lambda i: (0, 0)),
            pl.BlockSpec((F, 1), lambda i: (0, 0)),
            pl.BlockSpec((F, 1), lambda i: (0, 0)),
        ],
        out_specs=[
            pl.BlockSpec((RB, F), lambda i: (i, 0)),
            pl.BlockSpec((RB, 1), lambda i: (i, 0)),
            pl.BlockSpec((RB, 1), lambda i: (i, 0)),
        ],
        out_shape=[
            jax.ShapeDtypeStruct((N, F), jnp.float32),
            jax.ShapeDtypeStruct((N, 1), jnp.float32),
            jax.ShapeDtypeStruct((N, 1), jnp.float32),
        ],
    )(h, W, b.reshape(1, F), a1, a2)


def _sc_spmm(data_flat, src, dst, s1, s2):
    mesh = plsc.VectorSubcoreMesh(core_axis_name="c", subcore_axis_name="s",
                                  num_cores=NC, num_subcores=NS)

    @functools.partial(
        pl.kernel,
        out_type=jax.ShapeDtypeStruct((N, F), jnp.float32),
        mesh=mesh,
        compiler_params=pltpu.CompilerParams(needs_layout_passes=False,
                                             use_tc_tiling_on_sc=False),
        scratch_types=[
            pltpu.VMEM((N,), jnp.float32),
            pltpu.VMEM((N,), jnp.float32),
            pltpu.VMEM((BK,), jnp.int32),
            pltpu.VMEM((BK,), jnp.int32),
            pltpu.VMEM((2, NCH, CH), jnp.int32),
            pltpu.VMEM((2, NCH, CH), jnp.int32),
            pltpu.VMEM((2, NCH, CH), jnp.int32),
            pltpu.VMEM((2, BK), jnp.float32),
            pltpu.VMEM((2, BK, FG), jnp.float32),
            pltpu.SemaphoreType.DMA,
            pltpu.SemaphoreType.DMA,
            pltpu.SemaphoreType.DMA,
            [pltpu.SemaphoreType.DMA] * NCH,
            pltpu.VMEM_SHARED((ROWS_PER_SC, FG), jnp.float32),
            pltpu.VMEM_SHARED((2 * N,), jnp.float32),
        ],
    )
    def k(data_hbm, src_hbm, dst_hbm, s1_hbm, s2_hbm, out_hbm,
          s1_v, s2_v, src_v, dst_v, gidx_v, sidx_v, ridx_v, e_v, rows_v,
          sem_l, sem_s, sem_r, sem_g, acc_sh, rs_sh):
        c = lax.axis_index("c")
        s = lax.axis_index("s")
        gl = s // 2
        g = c * (G // NC) + gl
        half = s % 2
        e0 = half * HP

        zero16 = jnp.zeros((FG,), jnp.float32)
        _LANE = [jnp.full((16,), j, jnp.int32) for j in range(16)]

        pltpu.sync_copy(s1_hbm, s1_v)
        pltpu.sync_copy(s2_hbm, s2_v)

        def z_ab(j, _):
            rows_v[0, j, :] = zero16
            return 0
        lax.fori_loop(0, BK, z_ab, 0)
        zbase = gl * N + half * (N // 2)

        def z_acc(kk, _):
            pltpu.sync_copy(rows_v.at[0],
                            acc_sh.at[pl.ds(zbase + kk * BK, BK), :])
            return 0
        lax.fori_loop(0, (N // 2) // BK, z_acc, 0)
        pltpu.sync_copy(rows_v.at[0, pl.ds(0, (N // 2) % BK), :],
                        acc_sh.at[pl.ds(zbase + ((N // 2) // BK) * BK,
                                        (N // 2) % BK), :])

        @pl.when(gl == 0)
        def _z_rs():
            def z_e(j, _):
                e_v[0, pl.ds(j * 16, 16)] = zero16
                return 0
            lax.fori_loop(0, BK // 16, z_e, 0)

            def z_rsh(kk, _):
                pltpu.sync_copy(e_v.at[0],
                                rs_sh.at[pl.ds(half * N + kk * BK, BK)])
                return 0
            lax.fori_loop(0, N // BK, z_rsh, 0)
            pltpu.sync_copy(e_v.at[0, pl.ds(0, N % BK)],
                            rs_sh.at[pl.ds(half * N + (N // BK) * BK, N % BK)])

        plsc.subcore_barrier()

        def fire_loads(j):
            jc = jnp.minimum(j, NBLK - 1)
            base = e0 + jc * BK
            pltpu.async_copy(src_hbm.at[pl.ds(base, BK)], src_v, sem_l)
            pltpu.async_copy(dst_hbm.at[pl.ds(base, BK)], dst_v, sem_l)

        def drain_loads():
            pltpu.make_async_copy(src_hbm.at[pl.ds(0, BK)], src_v, sem_l).wait()
            pltpu.make_async_copy(dst_hbm.at[pl.ds(0, BK)], dst_v, sem_l).wait()

        def grp_compute(q):
            @plsc.parallel_loop(0, BK // 16, unroll=2)
            def _grp(i):
                sl = pl.ds(i * 16, 16)
                s16 = src_v[sl]
                d16 = dst_v[sl]
                sc = plsc.load_gather(s1_v, [s16]) + plsc.load_gather(s2_v, [d16])
                m = jnp.maximum(sc, sc * ALPHA)
                e16 = jnp.exp(m * INV_SQRT_F)
                e_v[q, sl] = e16
                kkq = i // 8
                lsl = pl.ds((i % 8) * 16, 16)
                gidx_v[q, kkq, lsl] = d16 + g * N
                sidx_v[q, kkq, lsl] = s16 + gl * N
                ridx_v[q, kkq, lsl] = s16 + half * N

        def fire_gathers(q):
            for kk in range(NCH):
                pltpu.async_copy(data_hbm.at[gidx_v.at[q, kk]],
                                 rows_v.at[q, pl.ds(kk * CH, CH), :],
                                 sem_g[kk])

        fire_loads(0)
        drain_loads()
        grp_compute(0)
        fire_loads(1)
        fire_gathers(0)

        def block(bi, _):
            p = bi % 2
            q = 1 - p
            drain_loads()
            grp_compute(q)
            fire_loads(bi + 2)

            for kk in range(NCH):
                pltpu.make_async_copy(
                    data_hbm.at[gidx_v.at[p, kk]],
                    rows_v.at[p, pl.ds(kk * CH, CH), :],
                    sem_g[kk]).wait()

                @plsc.parallel_loop(0, CH // 16, unroll=2)
                def _scale(i):
                    e16 = e_v[p, pl.ds(kk * CH + i * 16, 16)]
                    r0 = kk * CH + i * 16
                    for j2 in range(16):
                        mult = e16.at[_LANE[j2]].get(mode="promise_in_bounds")
                        rows_v[p, r0 + j2, :] = rows_v[p, r0 + j2, :] * mult

                pltpu.async_copy(rows_v.at[p, pl.ds(kk * CH, CH), :],
                                 acc_sh.at[sidx_v.at[p, kk]],
                                 sem_s, add=True)

                @pl.when(gl == 0)
                def _rs_add():
                    pltpu.async_copy(e_v.at[p, pl.ds(kk * CH, CH)],
                                     rs_sh.at[ridx_v.at[p, kk]],
                                     sem_r, add=True)

            @pl.when(bi < NBLK - 1)
            def _next_gathers():
                fire_gathers(q)

            for kk in range(NCH):
                pltpu.make_async_copy(rows_v.at[p, pl.ds(kk * CH, CH), :],
                                      acc_sh.at[sidx_v.at[p, kk]],
                                      sem_s).wait()

            @pl.when(gl == 0)
            def _rs_drain():
                for kk in range(NCH):
                    pltpu.make_async_copy(e_v.at[p, pl.ds(kk * CH, CH)],
                                          rs_sh.at[ridx_v.at[p, kk]],
                                          sem_r).wait()
            return 0
        lax.fori_loop(0, NBLK, block, 0)

        drain_loads()

        plsc.subcore_barrier()

        rbase0 = s * FIX_ROWS

        def fixblk(blk, _):
            rb = rbase0 + blk * FIX_BLK
            pltpu.sync_copy(acc_sh.at[pl.ds(rb, FIX_BLK), :],
                            rows_v.at[0, pl.ds(0, FIX_BLK), :])
            pltpu.sync_copy(data_hbm.at[pl.ds(c * ROWS_PER_SC + rb, FIX_BLK), :],
                            rows_v.at[1, pl.ds(0, FIX_BLK), :])
            nb = (rbase0 + blk * FIX_BLK) % N
            pltpu.sync_copy(rs_sh.at[pl.ds(nb, FIX_BLK)],
                            e_v.at[0, pl.ds(0, FIX_BLK)])
            pltpu.sync_copy(rs_sh.at[pl.ds(N + nb, FIX_BLK)],
                            e_v.at[1, pl.ds(0, FIX_BLK)])

            def rspass(i, _):
                sl = pl.ds(i * 16, 16)
                t = e_v[0, sl] + e_v[1, sl]
                iszero = t == 0.0
                den = jnp.where(iszero, 1.0, t)
                e_v[0, sl] = 1.0 / den
                e_v[1, sl] = jnp.where(iszero, 1.0, 0.0)
                return 0
            lax.fori_loop(0, RS_PAD // 16, rspass, 0)

            @plsc.parallel_loop(0, FIX_BLK // 16, unroll=2)
            def _rowfix(i):
                rcp16 = e_v[0, pl.ds(i * 16, 16)]
                m16 = e_v[1, pl.ds(i * 16, 16)]
                r0 = i * 16
                for j2 in range(16):
                    rcp = rcp16.at[_LANE[j2]].get(mode="promise_in_bounds")
                    m = m16.at[_LANE[j2]].get(mode="promise_in_bounds")
                    rows_v[0, r0 + j2, :] = (rows_v[0, r0 + j2, :]
                                             + m * rows_v[1, r0 + j2, :]) * rcp

            t0 = (FIX_BLK // 16) * 16
            rcp16 = e_v[0, pl.ds(t0, 16)]
            m16 = e_v[1, pl.ds(t0, 16)]
            for j2 in range(FIX_BLK - t0):
                rcp = rcp16.at[_LANE[j2]].get(mode="promise_in_bounds")
                m = m16.at[_LANE[j2]].get(mode="promise_in_bounds")
                rows_v[0, t0 + j2, :] = (rows_v[0, t0 + j2, :]
                                         + m * rows_v[1, t0 + j2, :]) * rcp
            pltpu.sync_copy(rows_v.at[0, pl.ds(0, FIX_BLK), :],
                            out_hbm.at[pl.ds(nb, FIX_BLK),
                                       pl.ds(g * FG, FG)])
            return 0
        lax.fori_loop(0, FIX_ROWS // FIX_BLK, fixblk, 0)

    return k(data_flat, src, dst, s1, s2)


def kernel(h, adj, W, b, a):
    src = adj[0].astype(jnp.int32)
    dst = adj[1].astype(jnp.int32)
    a1 = a[0, :F].reshape(F, 1)
    a2 = a[0, F:].reshape(F, 1)

    data, s1, s2 = _matmul_tc(h, W, b, a1, a2)

    data_flat = data.reshape(N, G, FG).transpose(1, 0, 2).reshape(G * N, FG)

    return _sc_spmm(data_flat, src, dst, s1.reshape(N), s2.reshape(N))


# speedup vs baseline: 1.1075x; 1.0166x over previous
import functools

import jax
import jax.numpy as jnp
from jax import lax
from jax.experimental import pallas as pl
from jax.experimental.pallas import tpu as pltpu
from jax.experimental.pallas import tpu_sc as plsc

N = 10000
E = 160000
F = 256
G = 16
FG = 16
NC = 2
NS = 16
HP = E // 2
CH = 128
NCH = 5
BK = CH * NCH
NBLK = HP // BK
ROWS_PER_SC = (G // NC) * N
FIX_ROWS = ROWS_PER_SC // NS
FIX_BLK = 200
RS_PAD = 208
ALPHA = 0.2
INV_SQRT_F = 1.0 / 16.0


def _matmul_tc(h, W, b, a1, a2):
    RB = 2000
    grid = (N // RB,)

    def body(h_ref, w_ref, b_ref, a1_ref, a2_ref, data_ref, s1_ref, s2_ref):
        dat = lax.dot_general(h_ref[...], w_ref[...],
                              (((1,), (1,)), ((), ())),
                              preferred_element_type=jnp.float32)
        dat = dat + b_ref[...]
        data_ref[...] = dat
        s1_ref[...] = lax.dot_general(dat, a1_ref[...],
                                      (((1,), (0,)), ((), ())),
                                      preferred_element_type=jnp.float32)
        s2_ref[...] = lax.dot_general(dat, a2_ref[...],
                                      (((1,), (0,)), ((), ())),
                                      preferred_element_type=jnp.float32)

    return pl.pallas_call(
        body,
        grid=grid,
        in_specs=[
            pl.BlockSpec((RB, F), lambda i: (i, 0)),
            pl.BlockSpec((F, F), lambda i: (0, 0)),
            pl.BlockSpec((1, F), lambda i: (0, 0)),
            pl.BlockSpec((F, 1), lambda i: (0, 0)),
            pl.BlockSpec((F, 1), lambda i: (0, 0)),
        ],
        out_specs=[
            pl.BlockSpec((RB, F), lambda i: (i, 0)),
            pl.BlockSpec((RB, 1), lambda i: (i, 0)),
            pl.BlockSpec((RB, 1), lambda i: (i, 0)),
        ],
        out_shape=[
            jax.ShapeDtypeStruct((N, F), jnp.float32),
            jax.ShapeDtypeStruct((N, 1), jnp.float32),
            jax.ShapeDtypeStruct((N, 1), jnp.float32),
        ],
    )(h, W, b.reshape(1, F), a1, a2)


def _sc_spmm(data_flat, src, dst, s1, s2):
    mesh = plsc.VectorSubcoreMesh(core_axis_name="c", subcore_axis_name="s",
                                  num_cores=NC, num_subcores=NS)

    @functools.partial(
        pl.kernel,
        out_type=jax.ShapeDtypeStruct((N, F), jnp.float32),
        mesh=mesh,
        compiler_params=pltpu.CompilerParams(needs_layout_passes=False,
                                             use_tc_tiling_on_sc=False),
        scratch_types=[
            pltpu.VMEM((N,), jnp.float32),
            pltpu.VMEM((N,), jnp.float32),
            pltpu.VMEM((BK,), jnp.int32),
            pltpu.VMEM((BK,), jnp.int32),
            pltpu.VMEM((2, NCH, CH), jnp.int32),
            pltpu.VMEM((2, NCH, CH), jnp.int32),
            pltpu.VMEM((2, NCH, CH), jnp.int32),
            pltpu.VMEM((2, BK), jnp.float32),
            pltpu.VMEM((2, BK, FG), jnp.float32),
            pltpu.SemaphoreType.DMA,
            pltpu.SemaphoreType.DMA,
            pltpu.SemaphoreType.DMA,
            [pltpu.SemaphoreType.DMA] * NCH,
            pltpu.VMEM_SHARED((ROWS_PER_SC, FG), jnp.float32),
            pltpu.VMEM_SHARED((2 * N,), jnp.float32),
        ],
    )
    def k(data_hbm, src_hbm, dst_hbm, s1_hbm, s2_hbm, out_hbm,
          s1_v, s2_v, src_v, dst_v, gidx_v, sidx_v, ridx_v, e_v, rows_v,
          sem_l, sem_s, sem_r, sem_g, acc_sh, rs_sh):
        c = lax.axis_index("c")
        s = lax.axis_index("s")
        gl = s // 2
        g = c * (G // NC) + gl
        half = s % 2
        e0 = half * HP

        zero16 = jnp.zeros((FG,), jnp.float32)
        _LANE = [jnp.full((16,), j, jnp.int32) for j in range(16)]

        pltpu.sync_copy(s1_hbm, s1_v)
        pltpu.sync_copy(s2_hbm, s2_v)

        def z_ab(j, _):
            rows_v[0, j, :] = zero16
            return 0
        lax.fori_loop(0, BK, z_ab, 0)
        zbase = gl * N + half * (N // 2)

        def z_acc(kk, _):
            pltpu.sync_copy(rows_v.at[0],
                            acc_sh.at[pl.ds(zbase + kk * BK, BK), :])
            return 0
        lax.fori_loop(0, (N // 2) // BK, z_acc, 0)
        pltpu.sync_copy(rows_v.at[0, pl.ds(0, (N // 2) % BK), :],
                        acc_sh.at[pl.ds(zbase + ((N // 2) // BK) * BK,
                                        (N // 2) % BK), :])

        @pl.when(gl == 0)
        def _z_rs():
            def z_e(j, _):
                e_v[0, pl.ds(j * 16, 16)] = zero16
                return 0
            lax.fori_loop(0, BK // 16, z_e, 0)

            def z_rsh(kk, _):
                pltpu.sync_copy(e_v.at[0],
                                rs_sh.at[pl.ds(half * N + kk * BK, BK)])
                return 0
            lax.fori_loop(0, N // BK, z_rsh, 0)
            pltpu.sync_copy(e_v.at[0, pl.ds(0, N % BK)],
                            rs_sh.at[pl.ds(half * N + (N // BK) * BK, N % BK)])

        plsc.subcore_barrier()

        def fire_loads(j):
            jc = jnp.minimum(j, NBLK - 1)
            base = e0 + jc * BK
            pltpu.async_copy(src_hbm.at[pl.ds(base, BK)], src_v, sem_l)
            pltpu.async_copy(dst_hbm.at[pl.ds(base, BK)], dst_v, sem_l)

        def drain_loads():
            pltpu.make_async_copy(src_hbm.at[pl.ds(0, BK)], src_v, sem_l).wait()
            pltpu.make_async_copy(dst_hbm.at[pl.ds(0, BK)], dst_v, sem_l).wait()

        def grp_compute(q):
            @plsc.parallel_loop(0, BK // 16, unroll=2)
            def _grp(i):
                sl = pl.ds(i * 16, 16)
                s16 = src_v[sl]
                d16 = dst_v[sl]
                sc = plsc.load_gather(s1_v, [s16]) + plsc.load_gather(s2_v, [d16])
                m = jnp.maximum(sc, sc * ALPHA)
                e16 = jnp.exp(m * INV_SQRT_F)
                e_v[q, sl] = e16
                kkq = i // 8
                lsl = pl.ds((i % 8) * 16, 16)
                gidx_v[q, kkq, lsl] = d16 + g * N
                sidx_v[q, kkq, lsl] = s16 + gl * N
                ridx_v[q, kkq, lsl] = s16 + half * N

        def fire_gathers(q):
            for kk in range(NCH):
                pltpu.async_copy(data_hbm.at[gidx_v.at[q, kk]],
                                 rows_v.at[q, pl.ds(kk * CH, CH), :],
                                 sem_g[kk])

        fire_loads(0)
        drain_loads()
        grp_compute(0)
        fire_loads(1)
        fire_gathers(0)

        def block(bi, _):
            p = bi % 2
            q = 1 - p
            drain_loads()
            grp_compute(q)
            fire_loads(bi + 2)

            for kk in range(NCH):
                pltpu.make_async_copy(
                    data_hbm.at[gidx_v.at[p, kk]],
                    rows_v.at[p, pl.ds(kk * CH, CH), :],
                    sem_g[kk]).wait()

                @plsc.parallel_loop(0, CH // 16, unroll=2)
                def _scale(i):
                    e16 = e_v[p, pl.ds(kk * CH + i * 16, 16)]
                    r0 = kk * CH + i * 16
                    for j2 in range(16):
                        mult = e16.at[_LANE[j2]].get(mode="promise_in_bounds")
                        rows_v[p, r0 + j2, :] = rows_v[p, r0 + j2, :] * mult

                pltpu.async_copy(rows_v.at[p, pl.ds(kk * CH, CH), :],
                                 acc_sh.at[sidx_v.at[p, kk]],
                                 sem_s, add=True)

                @pl.when(gl == 0)
                def _rs_add():
                    pltpu.async_copy(e_v.at[p, pl.ds(kk * CH, CH)],
                                     rs_sh.at[ridx_v.at[p, kk]],
                                     sem_r, add=True)

            @pl.when(bi < NBLK - 1)
            def _next_gathers():
                fire_gathers(q)

            for kk in range(NCH):
                pltpu.make_async_copy(rows_v.at[p, pl.ds(kk * CH, CH), :],
                                      acc_sh.at[sidx_v.at[p, kk]],
                                      sem_s).wait()

            @pl.when(gl == 0)
            def _rs_drain():
                for kk in range(NCH):
                    pltpu.make_async_copy(e_v.at[p, pl.ds(kk * CH, CH)],
                                          rs_sh.at[ridx_v.at[p, kk]],
                                          sem_r).wait()
            return 0
        lax.fori_loop(0, NBLK, block, 0)

        drain_loads()

        plsc.subcore_barrier()

        rbase0 = s * FIX_ROWS

        def fixblk(blk, _):
            rb = rbase0 + blk * FIX_BLK
            pltpu.sync_copy(acc_sh.at[pl.ds(rb, FIX_BLK), :],
                            rows_v.at[0, pl.ds(0, FIX_BLK), :])
            pltpu.sync_copy(data_hbm.at[pl.ds(c * ROWS_PER_SC + rb, FIX_BLK), :],
                            rows_v.at[1, pl.ds(0, FIX_BLK), :])
            nb = (rbase0 + blk * FIX_BLK) % N
            pltpu.sync_copy(rs_sh.at[pl.ds(nb, FIX_BLK)],
                            e_v.at[0, pl.ds(0, FIX_BLK)])
            pltpu.sync_copy(rs_sh.at[pl.ds(N + nb, FIX_BLK)],
                            e_v.at[1, pl.ds(0, FIX_BLK)])

            def rspass(i, _):
                sl = pl.ds(i * 16, 16)
                t = e_v[0, sl] + e_v[1, sl]
                iszero = t == 0.0
                den = jnp.where(iszero, 1.0, t)
                e_v[0, sl] = 1.0 / den
                e_v[1, sl] = jnp.where(iszero, 1.0, 0.0)
                return 0
            lax.fori_loop(0, RS_PAD // 16, rspass, 0)

            def _rowfix(i, _):
                rcp16 = e_v[0, pl.ds(i * 16, 16)]
                m16 = e_v[1, pl.ds(i * 16, 16)]
                r0 = i * 16
                for j2 in range(16):
                    rcp = rcp16.at[_LANE[j2]].get(mode="promise_in_bounds")
                    m = m16.at[_LANE[j2]].get(mode="promise_in_bounds")
                    rows_v[0, r0 + j2, :] = (rows_v[0, r0 + j2, :]
                                             + m * rows_v[1, r0 + j2, :]) * rcp
                return 0
            lax.fori_loop(0, FIX_BLK // 16, _rowfix, 0)

            t0 = (FIX_BLK // 16) * 16
            rcp16 = e_v[0, pl.ds(t0, 16)]
            m16 = e_v[1, pl.ds(t0, 16)]
            for j2 in range(FIX_BLK - t0):
                rcp = rcp16.at[_LANE[j2]].get(mode="promise_in_bounds")
                m = m16.at[_LANE[j2]].get(mode="promise_in_bounds")
                rows_v[0, t0 + j2, :] = (rows_v[0, t0 + j2, :]
                                         + m * rows_v[1, t0 + j2, :]) * rcp
            pltpu.sync_copy(rows_v.at[0, pl.ds(0, FIX_BLK), :],
                            out_hbm.at[pl.ds(nb, FIX_BLK),
                                       pl.ds(g * FG, FG)])
            return 0
        lax.fori_loop(0, FIX_ROWS // FIX_BLK, fixblk, 0)

    return k(data_flat, src, dst, s1, s2)


def kernel(h, adj, W, b, a):
    src = adj[0].astype(jnp.int32)
    dst = adj[1].astype(jnp.int32)
    a1 = a[0, :F].reshape(F, 1)
    a2 = a[0, F:].reshape(F, 1)

    data, s1, s2 = _matmul_tc(h, W, b, a1, a2)

    data_flat = data.reshape(N, G, FG).transpose(1, 0, 2).reshape(G * N, FG)

    return _sc_spmm(data_flat, src, dst, s1.reshape(N), s2.reshape(N))


# speedup vs baseline: 1.3636x; 1.2311x over previous
import functools

import jax
import jax.numpy as jnp
from jax import lax
from jax.experimental import pallas as pl
from jax.experimental.pallas import tpu as pltpu
from jax.experimental.pallas import tpu_sc as plsc

N = 10000
E = 160000
F = 256
G = 16
FG = 16
NC = 2
NS = 16
HP = E // 2
CH = 128
NCH = 5
BK = CH * NCH
NBLK = HP // BK
ROWS_PER_SC = (G // NC) * N
FIX_ROWS = ROWS_PER_SC // NS
FIX_BLK = 200
RS_PAD = 208
ALPHA = 0.2
INV_SQRT_F = 1.0 / 16.0


def _matmul_tc(h, W, b, a1, a2):
    RB = 2000
    grid = (N // RB,)

    def body(h_ref, w_ref, b_ref, a1_ref, a2_ref, data_ref, s1_ref, s2_ref):
        dat = lax.dot_general(h_ref[...], w_ref[...],
                              (((1,), (1,)), ((), ())),
                              preferred_element_type=jnp.float32)
        dat = dat + b_ref[...]
        data_ref[...] = dat
        s1_ref[...] = lax.dot_general(dat, a1_ref[...],
                                      (((1,), (0,)), ((), ())),
                                      preferred_element_type=jnp.float32)
        s2_ref[...] = lax.dot_general(dat, a2_ref[...],
                                      (((1,), (0,)), ((), ())),
                                      preferred_element_type=jnp.float32)

    return pl.pallas_call(
        body,
        grid=grid,
        in_specs=[
            pl.BlockSpec((RB, F), lambda i: (i, 0)),
            pl.BlockSpec((F, F), lambda i: (0, 0)),
            pl.BlockSpec((1, F), lambda i: (0, 0)),
            pl.BlockSpec((F, 1), lambda i: (0, 0)),
            pl.BlockSpec((F, 1), lambda i: (0, 0)),
        ],
        out_specs=[
            pl.BlockSpec((RB, F), lambda i: (i, 0)),
            pl.BlockSpec((RB, 1), lambda i: (i, 0)),
            pl.BlockSpec((RB, 1), lambda i: (i, 0)),
        ],
        out_shape=[
            jax.ShapeDtypeStruct((N, F), jnp.float32),
            jax.ShapeDtypeStruct((N, 1), jnp.float32),
            jax.ShapeDtypeStruct((N, 1), jnp.float32),
        ],
    )(h, W, b.reshape(1, F), a1, a2)


def _sc_spmm(data_flat, src, dst, s1, s2):
    mesh = plsc.VectorSubcoreMesh(core_axis_name="c", subcore_axis_name="s",
                                  num_cores=NC, num_subcores=NS)

    @functools.partial(
        pl.kernel,
        out_type=jax.ShapeDtypeStruct((N, F), jnp.float32),
        mesh=mesh,
        compiler_params=pltpu.CompilerParams(needs_layout_passes=False,
                                             use_tc_tiling_on_sc=False),
        scratch_types=[
            pltpu.VMEM((N,), jnp.float32),
            pltpu.VMEM((N,), jnp.float32),
            pltpu.VMEM((BK,), jnp.int32),
            pltpu.VMEM((BK,), jnp.int32),
            pltpu.VMEM((2, NCH, CH), jnp.int32),
            pltpu.VMEM((2, NCH, CH), jnp.int32),
            pltpu.VMEM((2, NCH, CH), jnp.int32),
            pltpu.VMEM((2, BK), jnp.float32),
            pltpu.VMEM((2, BK, FG), jnp.float32),
            pltpu.SemaphoreType.DMA,
            pltpu.SemaphoreType.DMA,
            pltpu.SemaphoreType.DMA,
            [pltpu.SemaphoreType.DMA] * NCH,
            pltpu.VMEM_SHARED((ROWS_PER_SC, FG), jnp.float32),
            pltpu.VMEM_SHARED((2 * N,), jnp.float32),
        ],
    )
    def k(data_hbm, src_hbm, dst_hbm, s1_hbm, s2_hbm, out_hbm,
          s1_v, s2_v, src_v, dst_v, gidx_v, sidx_v, ridx_v, e_v, rows_v,
          sem_l, sem_s, sem_r, sem_g, acc_sh, rs_sh):
        c = lax.axis_index("c")
        s = lax.axis_index("s")
        gl = s // 2
        g = c * (G // NC) + gl
        half = s % 2
        e0 = half * HP

        zero16 = jnp.zeros((FG,), jnp.float32)
        _LANE = [jnp.full((16,), j, jnp.int32) for j in range(16)]

        pltpu.sync_copy(s1_hbm, s1_v)
        pltpu.sync_copy(s2_hbm, s2_v)

        def z_ab(j, _):
            rows_v[0, j, :] = zero16
            return 0
        lax.fori_loop(0, BK, z_ab, 0)
        zbase = gl * N + half * (N // 2)

        def z_acc(kk, _):
            pltpu.sync_copy(rows_v.at[0],
                            acc_sh.at[pl.ds(zbase + kk * BK, BK), :])
            return 0
        lax.fori_loop(0, (N // 2) // BK, z_acc, 0)
        pltpu.sync_copy(rows_v.at[0, pl.ds(0, (N // 2) % BK), :],
                        acc_sh.at[pl.ds(zbase + ((N // 2) // BK) * BK,
                                        (N // 2) % BK), :])

        @pl.when(gl == 0)
        def _z_rs():
            def z_e(j, _):
                e_v[0, pl.ds(j * 16, 16)] = zero16
                return 0
            lax.fori_loop(0, BK // 16, z_e, 0)

            def z_rsh(kk, _):
                pltpu.sync_copy(e_v.at[0],
                                rs_sh.at[pl.ds(half * N + kk * BK, BK)])
                return 0
            lax.fori_loop(0, N // BK, z_rsh, 0)
            pltpu.sync_copy(e_v.at[0, pl.ds(0, N % BK)],
                            rs_sh.at[pl.ds(half * N + (N // BK) * BK, N % BK)])

        plsc.subcore_barrier()

        def fire_loads(j):
            jc = jnp.minimum(j, NBLK - 1)
            base = e0 + jc * BK
            pltpu.async_copy(src_hbm.at[pl.ds(base, BK)], src_v, sem_l)
            pltpu.async_copy(dst_hbm.at[pl.ds(base, BK)], dst_v, sem_l)

        def drain_loads():
            pltpu.make_async_copy(src_hbm.at[pl.ds(0, BK)], src_v, sem_l).wait()
            pltpu.make_async_copy(dst_hbm.at[pl.ds(0, BK)], dst_v, sem_l).wait()

        def grp_compute(q):
            @plsc.parallel_loop(0, BK // 16, unroll=2)
            def _grp(i):
                sl = pl.ds(i * 16, 16)
                s16 = src_v[sl]
                d16 = dst_v[sl]
                sc = plsc.load_gather(s1_v, [s16]) + plsc.load_gather(s2_v, [d16])
                m = jnp.maximum(sc, sc * ALPHA)
                e16 = jnp.exp(m * INV_SQRT_F)
                e_v[q, sl] = e16
                kkq = i // 8
                lsl = pl.ds((i % 8) * 16, 16)
                gidx_v[q, kkq, lsl] = d16 * G + g
                sidx_v[q, kkq, lsl] = s16 + gl * N
                ridx_v[q, kkq, lsl] = s16 + half * N

        def fire_gathers(q):
            for kk in range(NCH):
                pltpu.async_copy(data_hbm.at[gidx_v.at[q, kk]],
                                 rows_v.at[q, pl.ds(kk * CH, CH), :],
                                 sem_g[kk])

        fire_loads(0)
        drain_loads()
        grp_compute(0)
        fire_loads(1)
        fire_gathers(0)

        def block(bi, _):
            p = bi % 2
            q = 1 - p
            drain_loads()
            grp_compute(q)
            fire_loads(bi + 2)

            for kk in range(NCH):
                pltpu.make_async_copy(
                    data_hbm.at[gidx_v.at[p, kk]],
                    rows_v.at[p, pl.ds(kk * CH, CH), :],
                    sem_g[kk]).wait()

                @plsc.parallel_loop(0, CH // 16, unroll=2)
                def _scale(i):
                    e16 = e_v[p, pl.ds(kk * CH + i * 16, 16)]
                    r0 = kk * CH + i * 16
                    for j2 in range(16):
                        mult = e16.at[_LANE[j2]].get(mode="promise_in_bounds")
                        rows_v[p, r0 + j2, :] = rows_v[p, r0 + j2, :] * mult

                pltpu.async_copy(rows_v.at[p, pl.ds(kk * CH, CH), :],
                                 acc_sh.at[sidx_v.at[p, kk]],
                                 sem_s, add=True)

                @pl.when(gl == 0)
                def _rs_add():
                    pltpu.async_copy(e_v.at[p, pl.ds(kk * CH, CH)],
                                     rs_sh.at[ridx_v.at[p, kk]],
                                     sem_r, add=True)

            @pl.when(bi < NBLK - 1)
            def _next_gathers():
                fire_gathers(q)

            for kk in range(NCH):
                pltpu.make_async_copy(rows_v.at[p, pl.ds(kk * CH, CH), :],
                                      acc_sh.at[sidx_v.at[p, kk]],
                                      sem_s).wait()

            @pl.when(gl == 0)
            def _rs_drain():
                for kk in range(NCH):
                    pltpu.make_async_copy(e_v.at[p, pl.ds(kk * CH, CH)],
                                          rs_sh.at[ridx_v.at[p, kk]],
                                          sem_r).wait()
            return 0
        lax.fori_loop(0, NBLK, block, 0)

        drain_loads()

        plsc.subcore_barrier()

        rbase0 = s * FIX_ROWS

        iota16 = lax.iota(jnp.int32, 16)

        def fixblk(blk, _):
            rb = rbase0 + blk * FIX_BLK
            nb = (rbase0 + blk * FIX_BLK) % N
            d_acc = pltpu.async_copy(acc_sh.at[pl.ds(rb, FIX_BLK), :],
                                     rows_v.at[0, pl.ds(0, FIX_BLK), :], sem_l)
            d_ra = pltpu.async_copy(rs_sh.at[pl.ds(nb, FIX_BLK)],
                                    e_v.at[0, pl.ds(0, FIX_BLK)], sem_l)
            d_rb = pltpu.async_copy(rs_sh.at[pl.ds(N + nb, FIX_BLK)],
                                    e_v.at[1, pl.ds(0, FIX_BLK)], sem_l)

            def bidx(i, _):
                node16 = jnp.minimum(nb + i * 16 + iota16, N - 1)
                gidx_v[0, i // 8, pl.ds((i % 8) * 16, 16)] = node16 * G + g
                return 0
            lax.fori_loop(0, RS_PAD // 16, bidx, 0)
            d_g0 = pltpu.async_copy(data_hbm.at[gidx_v.at[0, 0]],
                                    rows_v.at[1, pl.ds(0, CH), :], sem_g[0])
            d_g1 = pltpu.async_copy(
                data_hbm.at[gidx_v.at[0, 1, pl.ds(0, RS_PAD - CH)]],
                rows_v.at[1, pl.ds(CH, RS_PAD - CH), :], sem_g[1])
            d_acc.wait()
            d_ra.wait()
            d_rb.wait()
            d_g0.wait()
            d_g1.wait()

            def rspass(i, _):
                sl = pl.ds(i * 16, 16)
                t = e_v[0, sl] + e_v[1, sl]
                iszero = t == 0.0
                den = jnp.where(iszero, 1.0, t)
                e_v[0, sl] = 1.0 / den
                e_v[1, sl] = jnp.where(iszero, 1.0, 0.0)
                return 0
            lax.fori_loop(0, RS_PAD // 16, rspass, 0)

            def _rowfix(i, _):
                rcp16 = e_v[0, pl.ds(i * 16, 16)]
                m16 = e_v[1, pl.ds(i * 16, 16)]
                r0 = i * 16
                for j2 in range(16):
                    rcp = rcp16.at[_LANE[j2]].get(mode="promise_in_bounds")
                    m = m16.at[_LANE[j2]].get(mode="promise_in_bounds")
                    rows_v[0, r0 + j2, :] = (rows_v[0, r0 + j2, :]
                                             + m * rows_v[1, r0 + j2, :]) * rcp
                return 0
            lax.fori_loop(0, FIX_BLK // 16, _rowfix, 0)

            t0 = (FIX_BLK // 16) * 16
            rcp16 = e_v[0, pl.ds(t0, 16)]
            m16 = e_v[1, pl.ds(t0, 16)]
            for j2 in range(FIX_BLK - t0):
                rcp = rcp16.at[_LANE[j2]].get(mode="promise_in_bounds")
                m = m16.at[_LANE[j2]].get(mode="promise_in_bounds")
                rows_v[0, t0 + j2, :] = (rows_v[0, t0 + j2, :]
                                         + m * rows_v[1, t0 + j2, :]) * rcp
            pltpu.sync_copy(rows_v.at[0, pl.ds(0, FIX_BLK), :],
                            out_hbm.at[pl.ds(nb, FIX_BLK),
                                       pl.ds(g * FG, FG)])
            return 0
        lax.fori_loop(0, FIX_ROWS // FIX_BLK, fixblk, 0)

    return k(data_flat, src, dst, s1, s2)


def kernel(h, adj, W, b, a):
    src = adj[0].astype(jnp.int32)
    dst = adj[1].astype(jnp.int32)
    a1 = a[0, :F].reshape(F, 1)
    a2 = a[0, F:].reshape(F, 1)

    data, s1, s2 = _matmul_tc(h, W, b, a1, a2)

    data_flat = data.reshape(N * G, FG)

    return _sc_spmm(data_flat, src, dst, s1.reshape(N), s2.reshape(N))
